# baseline (speedup 1.0000x reference)
import jax
import jax.numpy as jnp
from jax.experimental import pallas as pl
from jax.experimental.pallas import tpu as pltpu

_B = 8192
_DIN = 4096
_HID = 128
_E = 3
_CB = 256
_CBD = 32
_TB = 256
_NT = _B // _TB
_CH = 128
_NCH = _B // _CH

_F32 = jnp.float32


def _dot(a, b):
    return jax.lax.dot_general(a, b, (((a.ndim - 1,), (0,)), ((), ())),
                               preferred_element_type=_F32)


def _body(x_ref, w0, b0, w1, b1, w2, b2,
          win0, win1, win2, bin0, bin1, bin2,
          cb0, cb1, cb2, wout0, wout1, wout2, bout0, bout1, bout2,
          wu0, bu0, wu1, bu1, wu2, bu2,
          recon_ref, idxp_ref, dec_ref,
          qe0_scr, qe1_scr, qe2_scr, u_scr, cb2_scr, qs_scr):
    i = pl.program_id(0)
    wins = (win0, win1, win2)
    bins = (bin0, bin1, bin2)
    cbs = (cb0, cb1, cb2)
    wouts = (wout0, wout1, wout2)
    bouts = (bout0, bout1, bout2)

    @pl.when(i == 0)
    def _init():
        u_scr[...] = jnp.zeros((_E, _HID), _F32)
        for e in range(_E):
            cb = cbs[e][...]
            cb2_scr[e, 0:1, :] = (cb * cb).sum(-1)[None, :]

    _HB = _TB // 2
    scrs = (qe0_scr, qe1_scr, qe2_scr)
    for hh in range(2):
        rows = pl.ds(hh * _HB, _HB)
        xt = x_ref[rows, :]
        h = jnp.maximum(_dot(xt, w0[...]) + b0[...], 0.0)
        h = jnp.maximum(_dot(h, w1[...]) + b1[...], 0.0)
        lat = _dot(h, w2[...]) + b2[...]

        qe_sum = jnp.zeros((_HB, _HID), _F32)
        idxs = []
        for e in range(_E):
            zc = _dot(lat, wins[e][...]) + bins[e][...]
            d = ((zc * zc).sum(-1, keepdims=True)
                 - 2.0 * jax.lax.dot_general(
                     zc, cbs[e][...], (((1,), (1,)), ((), ())),
                     preferred_element_type=_F32)
                 + cb2_scr[e, 0:1, :])
            dmin = jnp.min(d, axis=-1, keepdims=True)
            iota = jax.lax.broadcasted_iota(jnp.int32, d.shape, 1)
            idx_e = jnp.min(jnp.where(d == dmin, iota, _CB),
                            axis=-1, keepdims=True)
            oh = (iota == idx_e).astype(_F32)
            q = _dot(oh, cbs[e][...])
            qe_e = _dot(q, wouts[e][...]) + bouts[e][...]
            scrs[e][pl.ds(i * _TB + hh * _HB, _HB), :] = qe_e
            u_scr[e, :] = u_scr[e, :] + qe_e.sum(axis=0)
            qe_sum = qe_sum + qe_e
            idxs.append(idx_e)

        colio = jax.lax.broadcasted_iota(jnp.int32, (_HB, _E), 1)
        idxp_ref[rows, :] = jnp.where(
            colio == 0, idxs[0],
            jnp.where(colio == 1, idxs[1], idxs[2]))

        rl = qe_sum / 3.0
        t = jnp.maximum(_dot(rl, wu0[...]) + bu0[...], 0.0)
        t = jnp.maximum(_dot(t, wu1[...]) + bu1[...], 0.0)
        r = _dot(t, wu2[...]) + bu2[...]
        recon_ref[rows, :] = jnp.clip(r, -1.0, 1.0)

    @pl.when(i == _NT - 1)
    def _finalize():
        m0 = u_scr[0, :] / _F32(_B)
        m1 = u_scr[1, :] / _F32(_B)
        m2 = u_scr[2, :] / _F32(_B)

        def _rd(scr, c, m):
            return scr[pl.ds(c * _CH, _CH), :] - m[None, :]

        def _chunk(c, acc):
            s00, s01, s02, s11, s12, s22 = acc
            c0 = _rd(qe0_scr, c, m0)
            c1 = _rd(qe1_scr, c, m1)
            c2 = _rd(qe2_scr, c, m2)
            s00 = s00 + jnp.sum(c0 * c0)
            s01 = s01 + jnp.sum(c0 * c1)
            s02 = s02 + jnp.sum(c0 * c2)
            s11 = s11 + jnp.sum(c1 * c1)
            s12 = s12 + jnp.sum(c1 * c2)
            s22 = s22 + jnp.sum(c2 * c2)
            return (s00, s01, s02, s11, s12, s22)

        z = _F32(0.0)
        s00, s01, s02, s11, s12, s22 = jax.lax.fori_loop(
            0, _NCH, _chunk, (z, z, z, z, z, z))
        denom = _F32(_B * _HID - 1)
        v00, v01, v02 = s00 / denom, s01 / denom, s02 / denom
        v11, v12, v22 = s11 / denom, s12 / denom, s22 / denom
        sd0 = jnp.sqrt(v00)
        sd1 = jnp.sqrt(v11)
        sd2 = jnp.sqrt(v22)
        sd0 = jnp.where(sd0 > 1e-8, sd0, _F32(1.0))
        sd1 = jnp.where(sd1 > 1e-8, sd1, _F32(1.0))
        sd2 = jnp.where(sd2 > 1e-8, sd2, _F32(1.0))
        c01 = v01 / (sd0 * sd1)
        c02 = v02 / (sd0 * sd2)
        c12 = v12 / (sd1 * sd2)
        dec = 2.0 * (c01 * c01 + c02 * c02 + c12 * c12)
        dec_ref[...] = jnp.full((8, _HID), dec, _F32)


def _full_spec(shape):
    nd = len(shape)
    return pl.BlockSpec(shape, lambda i, _n=nd: (0,) * _n)


def kernel(x, params):
    w0, w1, w2 = params["down_W"]
    b0, b1, b2 = [b.reshape(1, -1) for b in params["down_b"]]
    wu0, wu1, wu2 = params["up_W"]
    bu0, bu1, bu2 = [b.reshape(1, -1) for b in params["up_b"]]
    wins = params["proj_in_W"]
    bins = [b.reshape(1, -1) for b in params["proj_in_b"]]
    cbs = params["codebook"]
    wouts = params["proj_out_W"]
    bouts = [b.reshape(1, -1) for b in params["proj_out_b"]]

    args = (x, w0, b0, w1, b1, w2, b2,
            *wins, *bins, *cbs, *wouts, *bouts,
            wu0, bu0, wu1, bu1, wu2, bu2)
    in_specs = [pl.BlockSpec((_TB, _DIN), lambda i: (i, 0))]
    in_specs += [_full_spec(a.shape) for a in args[1:]]

    out_shape = (
        jax.ShapeDtypeStruct((_B, _DIN), _F32),
        jax.ShapeDtypeStruct((_B, _E), jnp.int32),
        jax.ShapeDtypeStruct((8, _HID), _F32),
    )
    out_specs = (
        pl.BlockSpec((_TB, _DIN), lambda i: (i, 0)),
        pl.BlockSpec((_TB, _E), lambda i: (i, 0)),
        _full_spec((8, _HID)),
    )
    recon, idxp, dec = pl.pallas_call(
        _body,
        grid=(_NT,),
        in_specs=in_specs,
        out_specs=out_specs,
        out_shape=out_shape,
        scratch_shapes=[
            pltpu.VMEM((_B, _HID), _F32),
            pltpu.VMEM((_B, _HID), _F32),
            pltpu.VMEM((_B, _HID), _F32),
            pltpu.VMEM((_E, _HID), _F32),
            pltpu.VMEM((_E, 8, _CB), _F32),
            pltpu.VMEM((2, _TB, _HID), _F32),
        ],
        compiler_params=pltpu.CompilerParams(
            dimension_semantics=("arbitrary",),
        ),
    )(*args)
    return recon, idxp, jnp.float32(0.0), dec[0, 0]


# speedup vs baseline: 1.4215x; 1.4215x over previous
import jax
import jax.numpy as jnp
from jax.experimental import pallas as pl
from jax.experimental.pallas import tpu as pltpu

_B = 8192
_DIN = 4096
_HID = 128
_E = 3
_CB = 256
_CBD = 32
_TB = 256
_NT = _B // _TB
_CH = 128
_NCH = _B // _CH

_F32 = jnp.float32


def _dot(a, b):
    return jax.lax.dot_general(a, b, (((a.ndim - 1,), (0,)), ((), ())),
                               preferred_element_type=_F32)


def _body(x_ref, w0, b0, w1, b1, w2, b2,
          win, bin_, cbs, wout, bout,
          wu0, bu0, wu1, bu1, wu2, bu2,
          recon_ref, idxp_ref, dec_ref,
          qe0_scr, qe1_scr, qe2_scr, u_scr, cb2_scr):
    i = pl.program_id(0)

    @pl.when(i == 0)
    def _init():
        u_scr[...] = jnp.zeros((_E, _HID), _F32)
        for e in range(_E):
            cb = cbs[e]
            cb2_scr[e, 0:1, :] = (cb * cb).sum(-1)[None, :]

    xt = x_ref[...]
    h = jnp.maximum(_dot(xt, w0[...]) + b0[...], 0.0)
    h = jnp.maximum(_dot(h, w1[...]) + b1[...], 0.0)
    lat = _dot(h, w2[...]) + b2[...]

    scrs = (qe0_scr, qe1_scr, qe2_scr)
    qe_sum = jnp.zeros((_TB, _HID), _F32)
    idxs = []
    for e in range(_E):
        zc = _dot(lat, win[e]) + bin_[e]
        d = ((zc * zc).sum(-1, keepdims=True)
             - 2.0 * jax.lax.dot_general(
                 zc, cbs[e], (((1,), (1,)), ((), ())),
                 preferred_element_type=_F32)
             + cb2_scr[e, 0:1, :])
        dmin = jnp.min(d, axis=-1, keepdims=True)
        iota = jax.lax.broadcasted_iota(jnp.int32, d.shape, 1)
        idx_e = jnp.min(jnp.where(d == dmin, iota, _CB),
                        axis=-1, keepdims=True)
        oh = (iota == idx_e).astype(_F32)
        q = _dot(oh, cbs[e])
        qe_e = _dot(q, wout[e]) + bout[e]
        scrs[e][pl.ds(i * _TB, _TB), :] = qe_e
        u_scr[e, :] = u_scr[e, :] + qe_e.sum(axis=0)
        qe_sum = qe_sum + qe_e
        idxs.append(idx_e)

    colio = jax.lax.broadcasted_iota(jnp.int32, (_TB, 8), 1)
    idxp_ref[...] = jnp.where(
        colio == 0, idxs[0],
        jnp.where(colio == 1, idxs[1],
                  jnp.where(colio == 2, idxs[2], 0)))

    rl = qe_sum / 3.0
    t = jnp.maximum(_dot(rl, wu0[...]) + bu0[...], 0.0)
    t = jnp.maximum(_dot(t, wu1[...]) + bu1[...], 0.0)
    r = _dot(t, wu2[...]) + bu2[...]
    recon_ref[...] = jnp.clip(r, -1.0, 1.0)

    @pl.when(i == _NT - 1)
    def _finalize():
        m0 = u_scr[0, :] / _F32(_B)
        m1 = u_scr[1, :] / _F32(_B)
        m2 = u_scr[2, :] / _F32(_B)

        def _rd(scr, c, m):
            return scr[pl.ds(c * _CH, _CH), :] - m[None, :]

        def _chunk(c, acc):
            s00, s01, s02, s11, s12, s22 = acc
            c0 = _rd(qe0_scr, c, m0)
            c1 = _rd(qe1_scr, c, m1)
            c2 = _rd(qe2_scr, c, m2)
            s00 = s00 + jnp.sum(c0 * c0)
            s01 = s01 + jnp.sum(c0 * c1)
            s02 = s02 + jnp.sum(c0 * c2)
            s11 = s11 + jnp.sum(c1 * c1)
            s12 = s12 + jnp.sum(c1 * c2)
            s22 = s22 + jnp.sum(c2 * c2)
            return (s00, s01, s02, s11, s12, s22)

        z = _F32(0.0)
        s00, s01, s02, s11, s12, s22 = jax.lax.fori_loop(
            0, _NCH, _chunk, (z, z, z, z, z, z))
        denom = _F32(_B * _HID - 1)
        v00, v01, v02 = s00 / denom, s01 / denom, s02 / denom
        v11, v12, v22 = s11 / denom, s12 / denom, s22 / denom
        sd0 = jnp.sqrt(v00)
        sd1 = jnp.sqrt(v11)
        sd2 = jnp.sqrt(v22)
        sd0 = jnp.where(sd0 > 1e-8, sd0, _F32(1.0))
        sd1 = jnp.where(sd1 > 1e-8, sd1, _F32(1.0))
        sd2 = jnp.where(sd2 > 1e-8, sd2, _F32(1.0))
        c01 = v01 / (sd0 * sd1)
        c02 = v02 / (sd0 * sd2)
        c12 = v12 / (sd1 * sd2)
        dec = 2.0 * (c01 * c01 + c02 * c02 + c12 * c12)
        dec_ref[...] = jnp.full((8, _HID), dec, _F32)


def _full_spec(shape):
    nd = len(shape)
    return pl.BlockSpec(shape, lambda i, _n=nd: (0,) * _n)


def kernel(x, params):
    w0, w1, w2 = params["down_W"]
    b0, b1, b2 = [b.reshape(1, -1) for b in params["down_b"]]
    wu0, wu1, wu2 = params["up_W"]
    bu0, bu1, bu2 = [b.reshape(1, -1) for b in params["up_b"]]
    win = jnp.stack(params["proj_in_W"])
    bin_ = jnp.stack([b.reshape(1, -1) for b in params["proj_in_b"]])
    cbs = jnp.stack(params["codebook"])
    wout = jnp.stack(params["proj_out_W"])
    bout = jnp.stack([b.reshape(1, -1) for b in params["proj_out_b"]])

    args = (x, w0, b0, w1, b1, w2, b2, win, bin_, cbs, wout, bout,
            wu0, bu0, wu1, bu1, wu2, bu2)
    in_specs = [pl.BlockSpec((_TB, _DIN), lambda i: (i, 0))]
    in_specs += [_full_spec(a.shape) for a in args[1:]]

    out_shape = (
        jax.ShapeDtypeStruct((_B, _DIN), _F32),
        jax.ShapeDtypeStruct((_B, 8), jnp.int32),
        jax.ShapeDtypeStruct((8, _HID), _F32),
    )
    out_specs = (
        pl.BlockSpec((_TB, _DIN), lambda i: (i, 0)),
        pl.BlockSpec((_TB, 8), lambda i: (i, 0)),
        _full_spec((8, _HID)),
    )
    recon, idxp, dec = pl.pallas_call(
        _body,
        grid=(_NT,),
        in_specs=in_specs,
        out_specs=out_specs,
        out_shape=out_shape,
        scratch_shapes=[
            pltpu.VMEM((_B, _HID), _F32),
            pltpu.VMEM((_B, _HID), _F32),
            pltpu.VMEM((_B, _HID), _F32),
            pltpu.VMEM((_E, _HID), _F32),
            pltpu.VMEM((_E, 8, _CB), _F32),
        ],
        compiler_params=pltpu.CompilerParams(
            dimension_semantics=("arbitrary",),
        ),
    )(*args)
    indices = idxp[:, :_E]
    return recon, indices, jnp.float32(0.0), dec[0, 0]
